# Initial kernel scaffold; baseline (speedup 1.0000x reference)
import jax
import jax.numpy as jnp
from jax.experimental import pallas as pl


def kernel(pred, target, label_weight):
    raise NotImplementedError("write your pallas kernel here")


# speedup vs baseline: 2.1855x; 2.1855x over previous
import functools

import jax
import jax.numpy as jnp
from jax import lax
from jax.experimental import pallas as pl
from jax.experimental.pallas import tpu as pltpu
from jax.experimental.pallas import tpu_sc as plsc

B = 4
C = 12
V = 262144
LN = 128
RW = V // LN
NW = 32
RPW = RW // NW
YT = 16
TPB = RPW // YT
NK = B * TPB
ZG = LN // 16

_LN2 = 0.6931471805599453
_SQRT2 = 1.4142135623730951


def _log16(s):
    bits = lax.bitcast_convert_type(s, jnp.int32)
    e = lax.shift_right_logical(bits, 23) - 127
    mbits = jnp.bitwise_or(jnp.bitwise_and(bits, 0x007FFFFF), 0x3F800000)
    m = lax.bitcast_convert_type(mbits, jnp.float32)
    big = m >= _SQRT2
    m = jnp.where(big, m * 0.5, m)
    e = e + jnp.where(big, 1, 0)
    t = (m - 1.0) / (m + 1.0)
    z = t * t
    p = t * (2.0 + z * (2.0 / 3.0 + z * (2.0 / 5.0 + z * (2.0 / 7.0 + z * (2.0 / 9.0)))))
    return e.astype(jnp.float32) * _LN2 + p


def _treemax(xs):
    while len(xs) > 1:
        xs = [jnp.maximum(xs[i], xs[i + 1]) if i + 1 < len(xs) else xs[i]
              for i in range(0, len(xs), 2)]
    return xs[0]


def _treesum(xs):
    while len(xs) > 1:
        xs = [xs[i] + xs[i + 1] if i + 1 < len(xs) else xs[i]
              for i in range(0, len(xs), 2)]
    return xs[0]


def _sc_body(pred_hbm, targ_hbm, lw_hbm, out_hbm,
             p_v0, p_v1, t_v0, t_v1, lw_v0, lw_v1, st_v, sems):
    w = lax.axis_index("s") * 2 + lax.axis_index("c")
    row0 = w * RPW
    bufs = ((p_v0, t_v0, lw_v0), (p_v1, t_v1, lw_v1))

    def tile_row(k):
        b = k // TPB
        row = row0 + (k % TPB) * YT
        return b, row

    def start_tile(k, slot):
        b, row = tile_row(k)
        pv, tv, lv = bufs[slot]
        pltpu.async_copy(pred_hbm.at[b, :, pl.ds(row, YT), :], pv, sems.at[slot])
        pltpu.async_copy(targ_hbm.at[b, :, pl.ds(row, YT), :], tv, sems.at[slot])
        pltpu.async_copy(lw_hbm.at[b, pl.ds(row, YT), :], lv, sems.at[slot])

    def wait_tile(slot):
        pv, tv, lv = bufs[slot]
        pltpu.make_async_copy(pred_hbm.at[0, :, pl.ds(0, YT), :], pv, sems.at[slot]).wait()
        pltpu.make_async_copy(targ_hbm.at[0, :, pl.ds(0, YT), :], tv, sems.at[slot]).wait()
        pltpu.make_async_copy(lw_hbm.at[0, pl.ds(0, YT), :], lv, sems.at[slot]).wait()

    def compute_tile(k, slot, carry):
        b, _ = tile_row(k)
        pv, tv, lv = bufs[slot]
        ind = [jnp.full((16,), (b == bb).astype(jnp.float32)) for bb in range(B)]

        def y_body(y, c2):
            accs, accc = c2
            accs, accc = list(accs), list(accc)
            for zg in range(ZG):
                sl = pl.ds(zg * 16, 16)
                p = [pv[c, y, sl] for c in range(C)]
                m = _treemax(p)
                s = _treesum([jnp.exp(pc - m) for pc in p])
                t = [tv[c, y, sl] for c in range(C)]
                ts = _treesum(t)
                tp = _treesum([tc * pc for tc, pc in zip(t, p)])
                lwv = lv[y, sl]
                valid = jnp.logical_and(lwv > 0, ts == 1.0)
                adds = jnp.where(valid, m + _log16(s) - tp, 0.0)
                addc = jnp.where(valid, 1.0, 0.0)
                for bb in range(B):
                    accs[bb] = accs[bb] + adds * ind[bb]
                    accc[bb] = accc[bb] + addc * ind[bb]
            return tuple(accs), tuple(accc)

        return lax.fori_loop(0, YT, y_body, carry)

    z16 = jnp.zeros((16,), jnp.float32)
    carry = ((z16,) * B, (z16,) * B)
    start_tile(0, 0)
    start_tile(1, 1)

    def k2_body(k2, carry):
        base = k2 * 2
        wait_tile(0)
        carry = compute_tile(base, 0, carry)

        @pl.when(base + 2 < NK)
        def _():
            start_tile(base + 2, 0)

        wait_tile(1)
        carry = compute_tile(base + 1, 1, carry)

        @pl.when(base + 3 < NK)
        def _():
            start_tile(base + 3, 1)

        return carry

    accs, accc = lax.fori_loop(0, NK // 2, k2_body, carry)

    for bb in range(B):
        st_v[pl.ds(0, 16)] = accs[bb]
        st_v[pl.ds(16, 16)] = accc[bb]
        pltpu.sync_copy(st_v, out_hbm.at[bb, w, :])


@functools.cache
def _build_sc_loss():
    return pl.kernel(
        _sc_body,
        out_type=jax.ShapeDtypeStruct((B, NW, LN), jnp.float32),
        mesh=plsc.VectorSubcoreMesh(core_axis_name="c", subcore_axis_name="s"),
        scratch_types=[
            pltpu.VMEM((C, YT, LN), jnp.float32),
            pltpu.VMEM((C, YT, LN), jnp.float32),
            pltpu.VMEM((C, YT, LN), jnp.float32),
            pltpu.VMEM((C, YT, LN), jnp.float32),
            pltpu.VMEM((YT, LN), jnp.int32),
            pltpu.VMEM((YT, LN), jnp.int32),
            pltpu.VMEM((LN,), jnp.float32),
            pltpu.SemaphoreType.DMA((2,)),
        ],
    )


def _combine(part_ref, out_ref):
    a = jnp.sum(part_ref[...], axis=1)
    s = jnp.sum(a[:, 0:16], axis=1, keepdims=True)
    c = jnp.sum(a[:, 16:32], axis=1, keepdims=True)
    per_b = jnp.where(c > 0, s / jnp.where(c > 0, c, 1.0), 0.0)
    out_ref[...] = jnp.sum(per_b, axis=0, keepdims=True) * (1.0 / B)


def kernel(pred, target, label_weight):
    p4 = pred.reshape(B, C, RW, LN)
    t4 = target.reshape(B, C, RW, LN)
    lw3 = label_weight.reshape(B, RW, LN)
    part = _build_sc_loss()(p4, t4, lw3)
    out = pl.pallas_call(
        _combine,
        out_shape=jax.ShapeDtypeStruct((1, 1), jnp.float32),
    )(part)
    return out[0, 0]


# speedup vs baseline: 5.0721x; 2.3207x over previous
import functools

import jax
import jax.numpy as jnp
from jax import lax
from jax.experimental import pallas as pl
from jax.experimental.pallas import tpu as pltpu
from jax.experimental.pallas import tpu_sc as plsc

B = 4
C = 12
X = 64
V = X * X * X
LN = 128
RW = V // LN
XCH = 8
GX = X // XCH
RPG = XCH * X * X // LN
NW = 32
RPW = RW // NW


def _tc_stage1(pred_ref, targ_ref, lw_ref, r_ref, v_ref):
    p = pred_ref[0]
    t = targ_ref[0]
    m = jnp.max(p, axis=0)
    s = jnp.sum(jnp.exp(p - m[None]), axis=0)
    lse = m + jnp.log(s)
    tp = jnp.sum(t * p, axis=0)
    ts = jnp.sum(t, axis=0)
    def to_rows(q):
        q4 = q.reshape(XCH, X // 2, 2, X)
        e = q4[:, :, 0, :]
        o = q4[:, :, 1, :]
        return jnp.concatenate([e, o], axis=-1).reshape(RPG, LN)

    kl = to_rows(lse - tp)
    tsv = to_rows(ts)
    lw = lw_ref[0]
    valid = jnp.logical_and(lw > 0, tsv == 1.0)
    r_ref[0] = jnp.where(valid, kl, 0.0)
    v_ref[0] = jnp.where(valid, 1.0, 0.0)


@functools.cache
def _build_stage1():
    return pl.pallas_call(
        _tc_stage1,
        grid=(B, GX),
        in_specs=[
            pl.BlockSpec((1, C, XCH, X, X), lambda b, g: (b, 0, g, 0, 0)),
            pl.BlockSpec((1, C, XCH, X, X), lambda b, g: (b, 0, g, 0, 0)),
            pl.BlockSpec((1, RPG, LN), lambda b, g: (b, g, 0)),
        ],
        out_specs=[
            pl.BlockSpec((1, RPG, LN), lambda b, g: (b, g, 0)),
            pl.BlockSpec((1, RPG, LN), lambda b, g: (b, g, 0)),
        ],
        out_shape=[
            jax.ShapeDtypeStruct((B, RW, LN), jnp.float32),
            jax.ShapeDtypeStruct((B, RW, LN), jnp.float32),
        ],
    )


def _sc_body(r_hbm, v_hbm, out_hbm, r_v0, r_v1, v_v0, v_v1, st_v, sems):
    w = lax.axis_index("s") * 2 + lax.axis_index("c")
    row0 = w * RPW
    bufs = ((r_v0, v_v0), (r_v1, v_v1))

    def start_tile(b, slot):
        rv, vv = bufs[slot]
        pltpu.async_copy(r_hbm.at[b, pl.ds(row0, RPW), :], rv, sems.at[slot])
        pltpu.async_copy(v_hbm.at[b, pl.ds(row0, RPW), :], vv, sems.at[slot])

    def wait_tile(slot):
        rv, vv = bufs[slot]
        pltpu.make_async_copy(r_hbm.at[0, pl.ds(0, RPW), :], rv, sems.at[slot]).wait()
        pltpu.make_async_copy(v_hbm.at[0, pl.ds(0, RPW), :], vv, sems.at[slot]).wait()

    start_tile(0, 0)
    start_tile(1, 1)
    for b in range(B):
        slot = b % 2
        wait_tile(slot)
        rv, vv = bufs[slot]

        def y_body(y, c2, rv=rv, vv=vv):
            s_acc, c_acc = c2
            for zg in range(LN // 16):
                sl = pl.ds(zg * 16, 16)
                s_acc = s_acc + rv[y, sl]
                c_acc = c_acc + vv[y, sl]
            return s_acc, c_acc

        acc = lax.fori_loop(
            0, RPW, y_body,
            (jnp.zeros((16,), jnp.float32), jnp.zeros((16,), jnp.float32)))

        if b + 2 < B:
            start_tile(b + 2, slot)
        st_v[pl.ds(0, 16)] = acc[0]
        st_v[pl.ds(16, 16)] = acc[1]
        pltpu.sync_copy(st_v, out_hbm.at[b, w, :])


@functools.cache
def _build_sc_reduce():
    return pl.kernel(
        _sc_body,
        out_type=jax.ShapeDtypeStruct((B, NW, LN), jnp.float32),
        mesh=plsc.VectorSubcoreMesh(core_axis_name="c", subcore_axis_name="s"),
        scratch_types=[
            pltpu.VMEM((RPW, LN), jnp.float32),
            pltpu.VMEM((RPW, LN), jnp.float32),
            pltpu.VMEM((RPW, LN), jnp.float32),
            pltpu.VMEM((RPW, LN), jnp.float32),
            pltpu.VMEM((LN,), jnp.float32),
            pltpu.SemaphoreType.DMA((2,)),
        ],
    )


def _combine(part_ref, out_ref):
    a = jnp.sum(part_ref[...], axis=1)
    s = jnp.sum(a[:, 0:16], axis=1, keepdims=True)
    c = jnp.sum(a[:, 16:32], axis=1, keepdims=True)
    per_b = jnp.where(c > 0, s / jnp.where(c > 0, c, 1.0), 0.0)
    out_ref[...] = jnp.sum(per_b, axis=0, keepdims=True) * (1.0 / B)


def kernel(pred, target, label_weight):
    lw3 = label_weight.reshape(B, RW, LN)
    r, v = _build_stage1()(pred, target, lw3)
    part = _build_sc_reduce()(r, v)
    out = pl.pallas_call(
        _combine,
        out_shape=jax.ShapeDtypeStruct((1, 1), jnp.float32),
    )(part)
    return out[0, 0]
